# baseline (speedup 1.0000x reference)
import dataclasses
import functools

import jax
import jax.numpy as jnp
from jax import lax
from jax.experimental import pallas as pl
from jax.experimental.pallas import tpu as pltpu
from jax.experimental.pallas import tpu_sc as plsc

NC, NS, L = 2, 16, 16
NW = NC * NS

_N = 10000
_E = 320000
_D = 128
_H = _D // NC

CH = 128
_EP = ((_E + NS * CH - 1) // (NS * CH)) * (NS * CH)
EWT = _EP // NS
NCH = EWT // CH
EWD = _EP // NW
GRP = 16
SR = _N // NS
_NPAD = _N + L

_VMESH = plsc.VectorSubcoreMesh(core_axis_name="c", subcore_axis_name="s")

_SC_PARAMS = pltpu.CompilerParams()
if "needs_layout_passes" in pltpu.CompilerParams.__dataclass_fields__:
    _SC_PARAMS = dataclasses.replace(_SC_PARAMS, needs_layout_passes=False)
_SC_UNTILED = dataclasses.replace(_SC_PARAMS, use_tc_tiling_on_sc=False)


def _deg_hist(dst_grouped):
    @functools.partial(
        pl.kernel,
        out_type=jax.ShapeDtypeStruct((NW, _NPAD), jnp.float32),
        mesh=_VMESH,
        compiler_params=_SC_PARAMS,
        scratch_types=[
            pltpu.VMEM((EWD,), jnp.int32),
            pltpu.VMEM((_NPAD,), jnp.float32),
        ],
    )
    def run(dst_hbm, hist_hbm, dstv, hist):
        cid = lax.axis_index("c")
        sid = lax.axis_index("s")
        wid = cid * NS + sid
        pltpu.sync_copy(dst_hbm.at[wid], dstv)
        zeros = jnp.zeros((L,), jnp.float32)

        @pl.loop(0, _NPAD, step=L)
        def _(i):
            hist.at[pl.ds(i, L)][...] = zeros

        ones = jnp.ones((L,), jnp.float32)

        @pl.loop(0, EWD, step=L)
        def _(i):
            idx = dstv[pl.ds(i, L)]
            plsc.addupdate_scatter(hist, [idx], ones)

        pltpu.sync_copy(hist, hist_hbm.at[wid])

    return run(dst_grouped)


def _aggregate(y2h, src3, dst3):
    @functools.partial(
        pl.kernel,
        out_type=jax.ShapeDtypeStruct((NC, NS, SR, _H), jnp.float32),
        mesh=_VMESH,
        compiler_params=_SC_UNTILED,
        scratch_types=[
            pltpu.VMEM((NCH, CH), jnp.int32),
            pltpu.VMEM((NCH, CH), jnp.int32),
            pltpu.VMEM((CH, _H), jnp.float32),
            pltpu.VMEM((CH, _H), jnp.float32),
            pltpu.VMEM_SHARED((_NPAD, _H), jnp.float32),
        ],
    )
    def run(ylo_hbm, yhi_hbm, src_hbm, dst_hbm, acc_hbm,
            srcv, dstv, rows0, rows1, acc_sh):
        cid = lax.axis_index("c")
        sid = lax.axis_index("s")
        pltpu.sync_copy(src_hbm.at[sid], srcv)
        pltpu.sync_copy(dst_hbm.at[sid], dstv)

        zeros = jnp.zeros((L,), jnp.float32)

        @pl.loop(0, CH)
        def _(i):
            @pl.loop(0, _H, step=L)
            def _(j):
                rows0.at[i, pl.ds(j, L)][...] = zeros

        for k in range(SR // CH):
            pltpu.sync_copy(rows0, acc_sh.at[pl.ds(sid * SR + k * CH, CH)])
        if SR % CH:
            pltpu.sync_copy(
                rows0.at[pl.ds(0, SR % CH)],
                acc_sh.at[pl.ds(sid * SR + (SR // CH) * CH, SR % CH)])

        @pl.when(sid == 0)
        def _():
            pltpu.sync_copy(rows0.at[pl.ds(0, _NPAD - _N)],
                            acc_sh.at[pl.ds(_N, _NPAD - _N)])

        plsc.subcore_barrier()

        def edge_loop(y_hbm):
            @pl.loop(0, NCH)
            def _(c):
                pltpu.sync_copy(y_hbm.at[srcv.at[c]], rows1)
                pltpu.sync_copy(rows1, acc_sh.at[dstv.at[c]], add=True)

        @pl.when(cid == 0)
        def _():
            edge_loop(ylo_hbm)

        @pl.when(cid == 1)
        def _():
            edge_loop(yhi_hbm)

        plsc.subcore_barrier()
        pltpu.sync_copy(acc_sh.at[pl.ds(sid * SR, SR)], acc_hbm.at[cid, sid])

    return run(y2h[0], y2h[1], src3, dst3).reshape(NC, _N, _H)


def _scale_rows(hist, x):
    def body(hist_ref, x_ref, y_ref):
        deg = jnp.sum(hist_ref[...][:, :_N], axis=0) + 1.0
        s = lax.rsqrt(deg)[:, None]
        y = x_ref[...] * s
        y_ref[0] = y[:, :_H]
        y_ref[1] = y[:, _H:]

    return pl.pallas_call(
        body,
        out_shape=jax.ShapeDtypeStruct((NC, x.shape[0], _H), jnp.float32),
    )(hist, x)


def _mid_dense(hist, x, acc, W1, b1, W2):
    def body(hist_ref, x_ref, acc_ref, W1_ref, b1_ref, W2_ref, y2_ref):
        deg = jnp.sum(hist_ref[...][:, :_N], axis=0) + 1.0
        s = lax.rsqrt(deg)[:, None]
        agg = jnp.concatenate([acc_ref[0], acc_ref[1]], axis=-1)
        pre = (agg + x_ref[...] * s) * s
        h1 = jnp.dot(pre, W1_ref[...],
                     preferred_element_type=jnp.float32,
                     precision=lax.Precision.HIGHEST)
        h1 = jnp.maximum(h1 + b1_ref[...][None, :], 0.0)
        g = jnp.dot(h1, W2_ref[...],
                    preferred_element_type=jnp.float32,
                    precision=lax.Precision.HIGHEST)
        y2 = g * s
        y2_ref[0] = y2[:, :_H]
        y2_ref[1] = y2[:, _H:]

    return pl.pallas_call(
        body,
        out_shape=jax.ShapeDtypeStruct((NC, x.shape[0], _H), jnp.float32),
    )(hist, x, acc, W1, b1, W2)


def _final_dense(hist, y2, acc, b2):
    def body(hist_ref, y2_ref, acc_ref, b2_ref, out_ref):
        deg = jnp.sum(hist_ref[...][:, :_N], axis=0) + 1.0
        s = lax.rsqrt(deg)[:, None]
        agg = jnp.concatenate([acc_ref[0], acc_ref[1]], axis=-1)
        y2 = jnp.concatenate([y2_ref[0], y2_ref[1]], axis=-1)
        out_ref[...] = (agg + y2) * s + b2_ref[...][None, :]

    return pl.pallas_call(
        body,
        out_shape=jax.ShapeDtypeStruct((y2.shape[1], _D), jnp.float32),
    )(hist, y2, acc, b2)


def kernel(x, edge_index, W1, b1, W2, b2):
    pad = _EP - _E
    src_p = jnp.concatenate(
        [edge_index[0], jnp.zeros((pad,), edge_index.dtype)])
    dst_p = jnp.concatenate(
        [edge_index[1], jnp.full((pad,), _N, edge_index.dtype)])
    src3 = src_p.reshape(NS, NCH, CH)
    dst3 = dst_p.reshape(NS, NCH, CH)
    dst2 = dst_p.reshape(NW, EWD)

    hist = _deg_hist(dst2)
    y1 = _scale_rows(hist, x)
    acc1 = _aggregate(y1, src3, dst3)
    y2 = _mid_dense(hist, x, acc1, W1, b1, W2)
    acc2 = _aggregate(y2, src3, dst3)
    return _final_dense(hist, y2, acc2, b2)


# speedup vs baseline: 1.0816x; 1.0816x over previous
import dataclasses
import functools

import jax
import jax.numpy as jnp
from jax import lax
from jax.experimental import pallas as pl
from jax.experimental.pallas import tpu as pltpu
from jax.experimental.pallas import tpu_sc as plsc

NC, NS, L = 2, 16, 16
NW = NC * NS

_N = 10000
_E = 320000
_D = 128
_H = _D // NC

CH = 128
_IGRAN = NS * CH * 16
_EP = ((_E + _IGRAN - 1) // _IGRAN) * _IGRAN
EWT = _EP // NS
NCH = EWT // CH
EWD = _EP // NW
IG = 16
NG = NCH // IG
SR = _N // NS
_NPAD = _N + L

_VMESH = plsc.VectorSubcoreMesh(core_axis_name="c", subcore_axis_name="s")

_SC_PARAMS = pltpu.CompilerParams()
if "needs_layout_passes" in pltpu.CompilerParams.__dataclass_fields__:
    _SC_PARAMS = dataclasses.replace(_SC_PARAMS, needs_layout_passes=False)
_SC_UNTILED = dataclasses.replace(_SC_PARAMS, use_tc_tiling_on_sc=False)


def _deg_hist(dst_grouped):
    @functools.partial(
        pl.kernel,
        out_type=jax.ShapeDtypeStruct((NW, _NPAD), jnp.float32),
        mesh=_VMESH,
        compiler_params=_SC_PARAMS,
        scratch_types=[
            pltpu.VMEM((EWD,), jnp.int32),
            pltpu.VMEM((_NPAD,), jnp.float32),
        ],
    )
    def run(dst_hbm, hist_hbm, dstv, hist):
        cid = lax.axis_index("c")
        sid = lax.axis_index("s")
        wid = cid * NS + sid
        pltpu.sync_copy(dst_hbm.at[wid], dstv)
        zeros = jnp.zeros((L,), jnp.float32)

        @pl.loop(0, _NPAD, step=L)
        def _(i):
            hist.at[pl.ds(i, L)][...] = zeros

        ones = jnp.ones((L,), jnp.float32)

        @pl.loop(0, EWD, step=L)
        def _(i):
            idx = dstv[pl.ds(i, L)]
            plsc.addupdate_scatter(hist, [idx], ones)

        pltpu.sync_copy(hist, hist_hbm.at[wid])

    return run(dst_grouped)


def _aggregate(y2h, src3, dst3):
    @functools.partial(
        pl.kernel,
        out_type=jax.ShapeDtypeStruct((NC, NS, SR, _H), jnp.float32),
        mesh=_VMESH,
        compiler_params=_SC_UNTILED,
        scratch_types=[
            pltpu.VMEM((IG, CH), jnp.int32),
            pltpu.VMEM((IG, CH), jnp.int32),
            pltpu.VMEM((CH, _H), jnp.float32),
            pltpu.VMEM_SHARED((_N, _H), jnp.float32),
            pltpu.VMEM_SHARED((_NPAD, _H), jnp.float32),
        ],
    )
    def run(ylo_hbm, yhi_hbm, src_hbm, dst_hbm, acc_hbm,
            srcv, dstv, rows, y_sh, acc_sh):
        cid = lax.axis_index("c")
        sid = lax.axis_index("s")

        @pl.when(cid == 0)
        def _():
            pltpu.sync_copy(ylo_hbm.at[pl.ds(sid * SR, SR)],
                            y_sh.at[pl.ds(sid * SR, SR)])

        @pl.when(cid == 1)
        def _():
            pltpu.sync_copy(yhi_hbm.at[pl.ds(sid * SR, SR)],
                            y_sh.at[pl.ds(sid * SR, SR)])

        zeros = jnp.zeros((L,), jnp.float32)

        @pl.loop(0, CH)
        def _(i):
            @pl.loop(0, _H, step=L)
            def _(j):
                rows.at[i, pl.ds(j, L)][...] = zeros

        for k in range(SR // CH):
            pltpu.sync_copy(rows, acc_sh.at[pl.ds(sid * SR + k * CH, CH)])
        if SR % CH:
            pltpu.sync_copy(
                rows.at[pl.ds(0, SR % CH)],
                acc_sh.at[pl.ds(sid * SR + (SR // CH) * CH, SR % CH)])

        @pl.when(sid == 0)
        def _():
            pltpu.sync_copy(rows.at[pl.ds(0, _NPAD - _N)],
                            acc_sh.at[pl.ds(_N, _NPAD - _N)])

        plsc.subcore_barrier()

        @pl.loop(0, NG)
        def _(g):
            pltpu.sync_copy(src_hbm.at[sid, g], srcv)
            pltpu.sync_copy(dst_hbm.at[sid, g], dstv)

            @pl.loop(0, IG)
            def _(j):
                pltpu.sync_copy(y_sh.at[srcv.at[j]], rows)
                pltpu.sync_copy(rows, acc_sh.at[dstv.at[j]], add=True)

        plsc.subcore_barrier()
        pltpu.sync_copy(acc_sh.at[pl.ds(sid * SR, SR)], acc_hbm.at[cid, sid])

    return run(y2h[0], y2h[1], src3, dst3).reshape(NC, _N, _H)


def _scale_rows(hist, x):
    def body(hist_ref, x_ref, y_ref):
        deg = jnp.sum(hist_ref[...][:, :_N], axis=0) + 1.0
        s = lax.rsqrt(deg)[:, None]
        y = x_ref[...] * s
        y_ref[0] = y[:, :_H]
        y_ref[1] = y[:, _H:]

    return pl.pallas_call(
        body,
        out_shape=jax.ShapeDtypeStruct((NC, x.shape[0], _H), jnp.float32),
    )(hist, x)


def _mid_dense(hist, x, acc, W1, b1, W2):
    def body(hist_ref, x_ref, acc_ref, W1_ref, b1_ref, W2_ref, y2_ref):
        deg = jnp.sum(hist_ref[...][:, :_N], axis=0) + 1.0
        s = lax.rsqrt(deg)[:, None]
        agg = jnp.concatenate([acc_ref[0], acc_ref[1]], axis=-1)
        pre = (agg + x_ref[...] * s) * s
        h1 = jnp.dot(pre, W1_ref[...],
                     preferred_element_type=jnp.float32,
                     precision=lax.Precision.HIGHEST)
        h1 = jnp.maximum(h1 + b1_ref[...][None, :], 0.0)
        g = jnp.dot(h1, W2_ref[...],
                    preferred_element_type=jnp.float32,
                    precision=lax.Precision.HIGHEST)
        y2 = g * s
        y2_ref[0] = y2[:, :_H]
        y2_ref[1] = y2[:, _H:]

    return pl.pallas_call(
        body,
        out_shape=jax.ShapeDtypeStruct((NC, x.shape[0], _H), jnp.float32),
    )(hist, x, acc, W1, b1, W2)


def _final_dense(hist, y2, acc, b2):
    def body(hist_ref, y2_ref, acc_ref, b2_ref, out_ref):
        deg = jnp.sum(hist_ref[...][:, :_N], axis=0) + 1.0
        s = lax.rsqrt(deg)[:, None]
        agg = jnp.concatenate([acc_ref[0], acc_ref[1]], axis=-1)
        y2 = jnp.concatenate([y2_ref[0], y2_ref[1]], axis=-1)
        out_ref[...] = (agg + y2) * s + b2_ref[...][None, :]

    return pl.pallas_call(
        body,
        out_shape=jax.ShapeDtypeStruct((y2.shape[1], _D), jnp.float32),
    )(hist, y2, acc, b2)


def kernel(x, edge_index, W1, b1, W2, b2):
    pad = _EP - _E
    src_p = jnp.concatenate(
        [edge_index[0], jnp.zeros((pad,), edge_index.dtype)])
    dst_p = jnp.concatenate(
        [edge_index[1], jnp.full((pad,), _N, edge_index.dtype)])
    src3 = src_p.reshape(NS, NG, IG, CH)
    dst3 = dst_p.reshape(NS, NG, IG, CH)
    dst2 = dst_p.reshape(NW, EWD)

    hist = _deg_hist(dst2)
    y1 = _scale_rows(hist, x)
    acc1 = _aggregate(y1, src3, dst3)
    y2 = _mid_dense(hist, x, acc1, W1, b1, W2)
    acc2 = _aggregate(y2, src3, dst3)
    return _final_dense(hist, y2, acc2, b2)
